# baseline (speedup 1.0000x reference)
import jax
import jax.numpy as jnp
import numpy as np
from jax import lax
from jax.experimental import pallas as pl
from jax.experimental.pallas import tpu as pltpu
from jax.experimental.pallas import tpu_sc as plsc

N = 10000
T = 1000
K_LOC = 5
K_REG = 15
N_CLUSTERS = 5
PERIODS = (0.25, 0.5, 1.0, 2.0)

N_SB = 10
N_CG = 3
CH_PER_G = 4
N_CH = 12
LANES = 16
NB = N // N_SB
N_CHUNKS = 63


def _prep_body(amp_t_ref, ph_t_ref, lab_t_ref, feat_ref, seg_ref):
    amp_t = amp_t_ref[:, :]
    ph_t = ph_t_ref[:, :]
    feat_t = jnp.concatenate(
        [amp_t, jnp.cos(ph_t), jnp.sin(ph_t), ph_t], axis=0)
    feat_ref[:, :] = feat_t
    lab = lab_t_ref[:, :]
    aug_t = jnp.concatenate([feat_t[:N_CH], jnp.ones((1, N), jnp.float32)],
                            axis=0)
    for c in range(N_CLUSTERS):
        m = (lab == c).astype(jnp.float32)
        seg_ref[c, :] = jnp.sum(aug_t * m, axis=1)


def _prep(amp_t, ph_t, lab_t):
    return pl.pallas_call(
        _prep_body,
        compiler_params=pltpu.CompilerParams(vmem_limit_bytes=62 * 2**20),
        out_shape=[
            jax.ShapeDtypeStruct((16, N), jnp.float32),
            jax.ShapeDtypeStruct((N_CLUSTERS, N_CH + 1), jnp.float32),
        ],
    )(amp_t, ph_t, lab_t)


def _mp_body(feat_hbm, lidx_hbm, lw_hbm, ridx_hbm, rw_hbm,
             loc_hbm, reg_hbm,
             feat_v, lidx_v, lw_v, ridx_v, rw_v, locacc, regacc):
    cid = lax.axis_index("c")
    sid = lax.axis_index("s")
    wid = sid * 2 + cid

    @pl.when(wid < N_SB * N_CG)
    def _work():
        cg = wid % N_CG
        sb = wid // N_CG
        base = sb * NB

        pltpu.sync_copy(feat_hbm.at[pl.ds(cg * CH_PER_G, CH_PER_G), :], feat_v)
        pltpu.sync_copy(lidx_hbm.at[pl.ds(base, NB), :], lidx_v)
        pltpu.sync_copy(lw_hbm.at[pl.ds(base, NB), :], lw_v)
        pltpu.sync_copy(ridx_hbm.at[pl.ds(base, NB), :], ridx_v)
        pltpu.sync_copy(rw_hbm.at[pl.ds(base, NB), :], rw_v)

        chv = [jnp.full((LANES,), ch, jnp.int32) for ch in range(CH_PER_G)]
        lkv = [jnp.full((LANES,), k, jnp.int32) for k in range(K_LOC)]
        rkv = [jnp.full((LANES,), k, jnp.int32) for k in range(K_REG)]
        lane = lax.iota(jnp.int32, LANES)

        def chunk(ci, _):
            off = jnp.minimum(ci * LANES, NB - LANES)
            sidv = lane + off
            livs = [plsc.load_gather(lidx_v, [sidv, lkv[k]]) for k in range(K_LOC)]
            lwvs = [plsc.load_gather(lw_v, [sidv, lkv[k]]) for k in range(K_LOC)]
            for ch in range(CH_PER_G):
                acc = jnp.zeros((LANES,), jnp.float32)
                for k in range(K_LOC):
                    acc = acc + lwvs[k] * plsc.load_gather(feat_v, [chv[ch], livs[k]])
                locacc[ch, pl.ds(off, LANES)] = acc
            rivs = [plsc.load_gather(ridx_v, [sidv, rkv[k]]) for k in range(K_REG)]
            rwvs = [plsc.load_gather(rw_v, [sidv, rkv[k]]) for k in range(K_REG)]
            for ch in range(CH_PER_G):
                acc = jnp.zeros((LANES,), jnp.float32)
                for k in range(K_REG):
                    acc = acc + rwvs[k] * plsc.load_gather(feat_v, [chv[ch], rivs[k]])
                regacc[ch, pl.ds(off, LANES)] = acc
            return 0

        lax.fori_loop(0, N_CHUNKS, chunk, 0)

        pltpu.sync_copy(locacc, loc_hbm.at[pl.ds(cg * CH_PER_G, CH_PER_G), pl.ds(base, NB)])
        pltpu.sync_copy(regacc, reg_hbm.at[pl.ds(cg * CH_PER_G, CH_PER_G), pl.ds(base, NB)])


def _message_pass(feat_t, lidx, lw, ridx, rw):
    mesh = plsc.VectorSubcoreMesh(
        core_axis_name="c", subcore_axis_name="s", num_cores=2, num_subcores=16)
    fn = pl.kernel(
        _mp_body,
        out_type=[
            jax.ShapeDtypeStruct((N_CH, N), jnp.float32),
            jax.ShapeDtypeStruct((N_CH, N), jnp.float32),
        ],
        mesh=mesh,
        compiler_params=pltpu.CompilerParams(
            use_tc_tiling_on_sc=False, needs_layout_passes=False),
        scratch_types=[
            pltpu.VMEM((CH_PER_G, N), jnp.float32),
            pltpu.VMEM((NB, K_LOC), jnp.int32),
            pltpu.VMEM((NB, K_LOC), jnp.float32),
            pltpu.VMEM((NB, K_REG), jnp.int32),
            pltpu.VMEM((NB, K_REG), jnp.float32),
            pltpu.VMEM((CH_PER_G, NB), jnp.float32),
            pltpu.VMEM((CH_PER_G, NB), jnp.float32),
        ],
    )
    return fn(feat_t, lidx, lw, ridx, rw)


TB = 200


def _signal_body(t_ref, feat_ref, loc_ref, reg_ref, lab_t_ref, seg_ref,
                 trend_ref, off_ref, out_ref, cmat_v):
    @pl.when(pl.program_id(0) == 0)
    def _build_c():
        lab = lab_t_ref[:, :]
        seg = seg_ref[:, :]
        clu = [jnp.zeros((1, N), jnp.float32) for _ in range(N_CH)]
        for c in range(N_CLUSTERS):
            sel = (lab == c).astype(jnp.float32)
            cnt = jnp.maximum(seg[c, N_CH], 1.0)
            for ch in range(N_CH):
                clu[ch] = clu[ch] + sel * (seg[c, ch] / cnt)
        rows = [off_ref[:, :], trend_ref[:, :]]
        arows = []
        brows = []
        for i in range(4):
            la = loc_ref[i:i + 1, :]
            lc = loc_ref[4 + i:5 + i, :]
            ls = loc_ref[8 + i:9 + i, :]
            ra = reg_ref[i:i + 1, :]
            rc = reg_ref[4 + i:5 + i, :]
            rs = reg_ref[8 + i:9 + i, :]
            amp_comb = 0.5 * la + 0.3 * ra + 0.2 * clu[i]
            ph_comb = (0.5 * jnp.arctan2(ls, lc)
                       + 0.3 * jnp.arctan2(rs, rc)
                       + 0.2 * jnp.arctan2(clu[8 + i], clu[4 + i]))
            amp_new = 0.7 * feat_ref[i:i + 1, :] + 0.3 * amp_comb
            ph_new = 0.7 * feat_ref[12 + i:13 + i, :] + 0.3 * ph_comb
            arows.append(amp_new * jnp.cos(ph_new))
            brows.append(amp_new * jnp.sin(ph_new))
        cmat_v[:, :] = jnp.concatenate(rows + arows + brows, axis=0)

    tcol = t_ref[:, :]
    c = cmat_v[:, :]
    cols = [jnp.ones((TB, 1), jnp.float32), tcol]
    cols += [jnp.sin((2.0 * np.pi / p) * tcol) for p in PERIODS]
    cols += [jnp.cos((2.0 * np.pi / p) * tcol) for p in PERIODS]
    basis = jnp.concatenate(cols, axis=1)
    out_ref[:, :] = jax.lax.dot_general(
        basis, c, (((1,), (0,)), ((), ())),
        preferred_element_type=jnp.float32,
        precision=jax.lax.Precision.HIGHEST)


def _signal(t_col, feat_t, loc_t, reg_t, lab_t, seg, trend, off):
    return pl.pallas_call(
        _signal_body,
        grid=(T // TB,),
        in_specs=[
            pl.BlockSpec((TB, 1), lambda i: (i, 0)),
            pl.BlockSpec((16, N), lambda i: (0, 0)),
            pl.BlockSpec((N_CH, N), lambda i: (0, 0)),
            pl.BlockSpec((N_CH, N), lambda i: (0, 0)),
            pl.BlockSpec((1, N), lambda i: (0, 0)),
            pl.BlockSpec((N_CLUSTERS, N_CH + 1), lambda i: (0, 0)),
            pl.BlockSpec((1, N), lambda i: (0, 0)),
            pl.BlockSpec((1, N), lambda i: (0, 0)),
        ],
        out_specs=pl.BlockSpec((TB, N), lambda i: (i, 0)),
        out_shape=jax.ShapeDtypeStruct((T, N), jnp.float32),
        scratch_shapes=[pltpu.VMEM((10, N), jnp.float32)],
        compiler_params=pltpu.CompilerParams(vmem_limit_bytes=62 * 2**20),
    )(t_col, feat_t, loc_t, reg_t, lab_t, seg, trend, off)


def kernel(time_vector, linear_trend, constant_offset, seasonal_amplitudes,
           seasonal_phases, spatial_adaptation_weights, local_idx, local_w,
           regional_idx, regional_w, cluster_labels):
    del spatial_adaptation_weights
    lab_t = cluster_labels.astype(jnp.int32).reshape(1, N)

    feat_t, seg = _prep(seasonal_amplitudes.T, seasonal_phases.T, lab_t)

    loc_t, reg_t = _message_pass(feat_t, local_idx.astype(jnp.int32), local_w,
                                 regional_idx.astype(jnp.int32), regional_w)

    out_t = _signal(time_vector.reshape(T, 1), feat_t, loc_t, reg_t, lab_t,
                    seg, linear_trend.reshape(1, N), constant_offset.reshape(1, N))
    return out_t.T


# speedup vs baseline: 1.2375x; 1.2375x over previous
import jax
import jax.numpy as jnp
import numpy as np
from jax import lax
from jax.experimental import pallas as pl
from jax.experimental.pallas import tpu as pltpu
from jax.experimental.pallas import tpu_sc as plsc

N = 10000
T = 1000
K_LOC = 5
K_REG = 15
N_CLUSTERS = 5
PERIODS = (0.25, 0.5, 1.0, 2.0)

N_SB = 10
N_CG = 3
CH_PER_G = 4
N_CH = 12
LANES = 16
NB = N // N_SB
N_CHUNKS = 63


def _prep_body(amp_t_ref, ph_t_ref, lab_t_ref, feat_ref, seg_ref):
    amp_t = amp_t_ref[:, :]
    ph_t = ph_t_ref[:, :]
    feat_t = jnp.concatenate(
        [amp_t, jnp.cos(ph_t), jnp.sin(ph_t), ph_t], axis=0)
    feat_ref[:, :] = feat_t
    lab = lab_t_ref[:, :]
    aug_t = jnp.concatenate([feat_t[:N_CH], jnp.ones((1, N), jnp.float32)],
                            axis=0)
    for c in range(N_CLUSTERS):
        m = (lab == c).astype(jnp.float32)
        seg_ref[c, :] = jnp.sum(aug_t * m, axis=1)


def _prep(amp_t, ph_t, lab_t):
    return pl.pallas_call(
        _prep_body,
        compiler_params=pltpu.CompilerParams(vmem_limit_bytes=62 * 2**20),
        out_shape=[
            jax.ShapeDtypeStruct((16, N), jnp.float32),
            jax.ShapeDtypeStruct((N_CLUSTERS, N_CH + 1), jnp.float32),
        ],
    )(amp_t, ph_t, lab_t)


def _mp_body(feat_hbm, lidx_hbm, lw_hbm, ridx_hbm, rw_hbm,
             loc_hbm, reg_hbm,
             feat_v, lidx_v, lw_v, ridx_v, rw_v, locacc, regacc):
    cid = lax.axis_index("c")
    sid = lax.axis_index("s")
    wid = sid * 2 + cid

    @pl.when(wid < N_SB * N_CG)
    def _work():
        cg = wid % N_CG
        sb = wid // N_CG
        base = sb * NB

        pltpu.sync_copy(feat_hbm.at[pl.ds(cg * CH_PER_G, CH_PER_G), :], feat_v)
        pltpu.sync_copy(lidx_hbm.at[pl.ds(base, NB), :], lidx_v)
        pltpu.sync_copy(lw_hbm.at[pl.ds(base, NB), :], lw_v)
        pltpu.sync_copy(ridx_hbm.at[pl.ds(base, NB), :], ridx_v)
        pltpu.sync_copy(rw_hbm.at[pl.ds(base, NB), :], rw_v)

        chv = [jnp.full((LANES,), ch, jnp.int32) for ch in range(CH_PER_G)]
        lkv = [jnp.full((LANES,), k, jnp.int32) for k in range(K_LOC)]
        rkv = [jnp.full((LANES,), k, jnp.int32) for k in range(K_REG)]
        lane = lax.iota(jnp.int32, LANES)

        def chunk(ci, _):
            off = jnp.minimum(ci * LANES, NB - LANES)
            sidv = lane + off
            livs = [plsc.load_gather(lidx_v, [sidv, lkv[k]]) for k in range(K_LOC)]
            lwvs = [plsc.load_gather(lw_v, [sidv, lkv[k]]) for k in range(K_LOC)]
            for ch in range(CH_PER_G):
                acc = jnp.zeros((LANES,), jnp.float32)
                for k in range(K_LOC):
                    acc = acc + lwvs[k] * plsc.load_gather(feat_v, [chv[ch], livs[k]])
                locacc[ch, pl.ds(off, LANES)] = acc
            rivs = [plsc.load_gather(ridx_v, [sidv, rkv[k]]) for k in range(K_REG)]
            rwvs = [plsc.load_gather(rw_v, [sidv, rkv[k]]) for k in range(K_REG)]
            for ch in range(CH_PER_G):
                acc = jnp.zeros((LANES,), jnp.float32)
                for k in range(K_REG):
                    acc = acc + rwvs[k] * plsc.load_gather(feat_v, [chv[ch], rivs[k]])
                regacc[ch, pl.ds(off, LANES)] = acc
            return 0

        lax.fori_loop(0, N_CHUNKS, chunk, 0)

        pltpu.sync_copy(locacc, loc_hbm.at[pl.ds(cg * CH_PER_G, CH_PER_G), pl.ds(base, NB)])
        pltpu.sync_copy(regacc, reg_hbm.at[pl.ds(cg * CH_PER_G, CH_PER_G), pl.ds(base, NB)])


def _message_pass(feat_t, lidx, lw, ridx, rw):
    mesh = plsc.VectorSubcoreMesh(
        core_axis_name="c", subcore_axis_name="s", num_cores=2, num_subcores=16)
    fn = pl.kernel(
        _mp_body,
        out_type=[
            jax.ShapeDtypeStruct((N_CH, N), jnp.float32),
            jax.ShapeDtypeStruct((N_CH, N), jnp.float32),
        ],
        mesh=mesh,
        compiler_params=pltpu.CompilerParams(
            use_tc_tiling_on_sc=False, needs_layout_passes=False),
        scratch_types=[
            pltpu.VMEM((CH_PER_G, N), jnp.float32),
            pltpu.VMEM((NB, K_LOC), jnp.int32),
            pltpu.VMEM((NB, K_LOC), jnp.float32),
            pltpu.VMEM((NB, K_REG), jnp.int32),
            pltpu.VMEM((NB, K_REG), jnp.float32),
            pltpu.VMEM((CH_PER_G, NB), jnp.float32),
            pltpu.VMEM((CH_PER_G, NB), jnp.float32),
        ],
    )
    return fn(feat_t, lidx, lw, ridx, rw)


TB = 200


def _signal_body(t_ref, feat_ref, loc_ref, reg_ref, lab_t_ref, seg_ref,
                 trend_ref, off_ref, out_ref, cmat_v):
    @pl.when(pl.program_id(0) == 0)
    def _build_c():
        lab = lab_t_ref[:, :]
        seg = seg_ref[:, :]
        clu = [jnp.zeros((1, N), jnp.float32) for _ in range(N_CH)]
        for c in range(N_CLUSTERS):
            sel = (lab == c).astype(jnp.float32)
            cnt = jnp.maximum(seg[c, N_CH], 1.0)
            for ch in range(N_CH):
                clu[ch] = clu[ch] + sel * (seg[c, ch] / cnt)
        rows = [off_ref[:, :], trend_ref[:, :]]
        arows = []
        brows = []
        for i in range(4):
            la = loc_ref[i:i + 1, :]
            lc = loc_ref[4 + i:5 + i, :]
            ls = loc_ref[8 + i:9 + i, :]
            ra = reg_ref[i:i + 1, :]
            rc = reg_ref[4 + i:5 + i, :]
            rs = reg_ref[8 + i:9 + i, :]
            amp_comb = 0.5 * la + 0.3 * ra + 0.2 * clu[i]
            ph_comb = (0.5 * jnp.arctan2(ls, lc)
                       + 0.3 * jnp.arctan2(rs, rc)
                       + 0.2 * jnp.arctan2(clu[8 + i], clu[4 + i]))
            amp_new = 0.7 * feat_ref[i:i + 1, :] + 0.3 * amp_comb
            ph_new = 0.7 * feat_ref[12 + i:13 + i, :] + 0.3 * ph_comb
            arows.append(amp_new * jnp.cos(ph_new))
            brows.append(amp_new * jnp.sin(ph_new))
        cmat_v[:, :] = jnp.concatenate(rows + arows + brows, axis=0)

    tcol = t_ref[:, :]
    c = cmat_v[:, :]
    cols = [jnp.ones((TB, 1), jnp.float32), tcol]
    cols += [jnp.sin((2.0 * np.pi / p) * tcol) for p in PERIODS]
    cols += [jnp.cos((2.0 * np.pi / p) * tcol) for p in PERIODS]
    basis = jnp.concatenate(cols, axis=1)
    out_ref[:, :] = jax.lax.dot_general(
        basis, c, (((1,), (0,)), ((), ())),
        preferred_element_type=jnp.float32)


def _signal(t_col, feat_t, loc_t, reg_t, lab_t, seg, trend, off):
    return pl.pallas_call(
        _signal_body,
        grid=(T // TB,),
        in_specs=[
            pl.BlockSpec((TB, 1), lambda i: (i, 0)),
            pl.BlockSpec((16, N), lambda i: (0, 0)),
            pl.BlockSpec((N_CH, N), lambda i: (0, 0)),
            pl.BlockSpec((N_CH, N), lambda i: (0, 0)),
            pl.BlockSpec((1, N), lambda i: (0, 0)),
            pl.BlockSpec((N_CLUSTERS, N_CH + 1), lambda i: (0, 0)),
            pl.BlockSpec((1, N), lambda i: (0, 0)),
            pl.BlockSpec((1, N), lambda i: (0, 0)),
        ],
        out_specs=pl.BlockSpec((TB, N), lambda i: (i, 0)),
        out_shape=jax.ShapeDtypeStruct((T, N), jnp.float32),
        scratch_shapes=[pltpu.VMEM((10, N), jnp.float32)],
        compiler_params=pltpu.CompilerParams(vmem_limit_bytes=62 * 2**20),
    )(t_col, feat_t, loc_t, reg_t, lab_t, seg, trend, off)


def kernel(time_vector, linear_trend, constant_offset, seasonal_amplitudes,
           seasonal_phases, spatial_adaptation_weights, local_idx, local_w,
           regional_idx, regional_w, cluster_labels):
    del spatial_adaptation_weights
    lab_t = cluster_labels.astype(jnp.int32).reshape(1, N)

    feat_t, seg = _prep(seasonal_amplitudes.T, seasonal_phases.T, lab_t)

    loc_t, reg_t = _message_pass(feat_t, local_idx.astype(jnp.int32), local_w,
                                 regional_idx.astype(jnp.int32), regional_w)

    out_t = _signal(time_vector.reshape(T, 1), feat_t, loc_t, reg_t, lab_t,
                    seg, linear_trend.reshape(1, N), constant_offset.reshape(1, N))
    return out_t.T
